# baseline (speedup 1.0000x reference)
import functools

import jax
import jax.numpy as jnp
from jax import lax
from jax.experimental import pallas as pl
from jax.experimental.pallas import tpu as pltpu
from jax.experimental.pallas import tpu_sc as plsc

ROWS = 1024
COLS = 100000
K = 64
N = ROWS * K

_info = plsc.get_sparse_core_info()
NC = _info.num_cores
NS = _info.num_subcores
NW = NC * NS
RPW = ROWS // NW
EPW = N // NW
SPW = RPW // 8

CT = 49
CW = CT * 128
NCK = 16
TMAX = (COLS + 127) // 128 - CT
NCH = SPW * NCK

_mesh = plsc.VectorSubcoreMesh(core_axis_name="c", subcore_axis_name="s")


@functools.partial(
    pl.kernel,
    mesh=_mesh,
    compiler_params=pltpu.CompilerParams(use_tc_tiling_on_sc=False),
    out_type=jax.ShapeDtypeStruct((N,), jnp.float32),
    scratch_types=[
        pltpu.VMEM((EPW,), jnp.int32),
        pltpu.VMEM((EPW,), jnp.float32),
        pltpu.VMEM((RPW, 16), jnp.float32),
        pltpu.VMEM_SHARED((NS, 2, 8, CW), jnp.float32),
        pltpu.SemaphoreType.DMA,
        pltpu.SemaphoreType.DMA,
        pltpu.SemaphoreType.DMA,
    ],
)
def _gather_kernel(
    x_hbm, idx_hbm, out_hbm, idx_v, out_v, tmp_v, spm, sem_a, sem_b, gsem
):
    cid = lax.axis_index("c")
    tec = lax.axis_index("s")
    wid = tec * NC + cid
    ebase = wid * EPW
    rowbase = wid * RPW
    pltpu.sync_copy(idx_hbm.at[pl.ds(ebase, EPW)], idx_v)

    zeros16 = jnp.zeros((16,), jnp.float32)

    @pl.loop(0, EPW // 16)
    def _init(g):
        out_v[pl.ds(g * 16, 16)] = zeros16

    def fire(n, par, sem):
        stripe = n >> 4
        t0 = jnp.minimum((n & 15) * CT, TMAX)
        row8 = pl.multiple_of(rowbase + stripe * 8, 8)
        c0 = pl.multiple_of(t0 * 128, 128)
        pltpu.async_copy(
            x_hbm.at[pl.ds(row8, 8), pl.ds(c0, CW)], spm.at[tec, par], sem
        )

    def drain(par, sem):
        pltpu.make_async_copy(
            x_hbm.at[pl.ds(0, 8), pl.ds(0, CW)], spm.at[tec, par], sem
        ).wait()

    def extract(n, par):
        stripe = n >> 4
        col0 = jnp.minimum((n & 15) * CT, TMAX) * 128
        gbase = stripe * 32
        cps = []
        for gg in range(32):
            subl = (gg >> 2) & 7
            jv = idx_v[pl.ds((gbase + gg) * 16, 16)]
            loc = jnp.clip(jv - col0, 0, CW - 1)
            cps.append(
                pltpu.async_copy(
                    spm.at[tec, par, subl].at[loc], tmp_v.at[gg], gsem
                )
            )
        for cp in cps:
            cp.wait()
        for gg in range(32):
            g = gbase + gg
            jv = idx_v[pl.ds(g * 16, 16)]
            inb = (jv >= col0) & (jv < col0 + CW)
            acc = out_v[pl.ds(g * 16, 16)]
            out_v[pl.ds(g * 16, 16)] = jnp.where(inb, tmp_v[gg], acc)

    fire(0, 0, sem_a)
    fire(1, 1, sem_b)

    @pl.loop(0, NCH // 2 - 1)
    def _pipeline(k):
        n0 = 2 * k
        drain(0, sem_a)
        extract(n0, 0)
        fire(n0 + 2, 0, sem_a)
        drain(1, sem_b)
        extract(n0 + 1, 1)
        fire(n0 + 3, 1, sem_b)

    drain(0, sem_a)
    extract(NCH - 2, 0)
    drain(1, sem_b)
    extract(NCH - 1, 1)

    pltpu.sync_copy(out_v, out_hbm.at[pl.ds(ebase, EPW)])


def kernel(x, index):
    out = _gather_kernel(x, index.reshape(N))
    return out.reshape(ROWS, K)


# speedup vs baseline: 2.0095x; 2.0095x over previous
import functools

import jax
import jax.numpy as jnp
from jax import lax
from jax.experimental import pallas as pl
from jax.experimental.pallas import tpu as pltpu
from jax.experimental.pallas import tpu_sc as plsc

ROWS = 1024
COLS = 100000
K = 64
N = ROWS * K

_info = plsc.get_sparse_core_info()
NC = _info.num_cores
NS = _info.num_subcores
NW = NC * NS
RPW = ROWS // NW
EPW = N // NW
SPW = RPW // 8

CT = 49
CW = CT * 128
NCK = 16
TMAX = (COLS + 127) // 128 - CT
NCH = SPW * NCK

_GDN = lax.GatherDimensionNumbers(
    offset_dims=(), collapsed_slice_dims=(0,), start_index_map=(0,)
)

_mesh = plsc.VectorSubcoreMesh(core_axis_name="c", subcore_axis_name="s")


@functools.partial(
    pl.kernel,
    mesh=_mesh,
    out_type=jax.ShapeDtypeStruct((N,), jnp.float32),
    scratch_types=[
        pltpu.VMEM((EPW,), jnp.int32),
        pltpu.VMEM((EPW,), jnp.float32),
        pltpu.VMEM((8, CW), jnp.float32),
        pltpu.VMEM((8, CW), jnp.float32),
        pltpu.SemaphoreType.DMA,
        pltpu.SemaphoreType.DMA,
    ],
)
def _gather_kernel(
    x_hbm, idx_hbm, out_hbm, idx_v, out_v, buf_a, buf_b, sem_a, sem_b
):
    wid = lax.axis_index("s") * NC + lax.axis_index("c")
    ebase = wid * EPW
    rowbase = wid * RPW
    pltpu.sync_copy(idx_hbm.at[pl.ds(ebase, EPW)], idx_v)

    zeros16 = jnp.zeros((16,), jnp.float32)

    @pl.loop(0, EPW // 16)
    def _init(g):
        out_v[pl.ds(g * 16, 16)] = zeros16

    lanes16 = lax.iota(jnp.int32, 16)

    def fire(n, buf, sem):
        stripe = n >> 4
        t0 = jnp.minimum((n & 15) * CT, TMAX)
        row8 = pl.multiple_of(rowbase + stripe * 8, 8)
        c0 = pl.multiple_of(t0 * 128, 128)
        pltpu.async_copy(
            x_hbm.at[pl.ds(row8, 8), pl.ds(c0, CW)], buf, sem
        )

    def drain(buf, sem):
        pltpu.make_async_copy(
            x_hbm.at[pl.ds(0, 8), pl.ds(0, CW)], buf, sem
        ).wait()

    def extract(n, buf):
        stripe = n >> 4
        col0 = jnp.minimum((n & 15) * CT, TMAX) * 128
        gbase = stripe * 32

        @pl.loop(0, 32)
        def _group(gg):
            subl = (gg >> 2) & 7
            g = gbase + gg
            jv = idx_v[pl.ds(g * 16, 16)]
            for t in range(16):
                s = jv[t]
                loc = s - col0

                @pl.when((loc >= 0) & (loc < CW))
                def _do_elem():
                    v2 = buf[
                        subl, pl.ds(pl.multiple_of(loc & -16, 16), 16)
                    ]
                    w = lax.gather(
                        v2,
                        jnp.full((16,), loc & 15, jnp.int32)[:, None],
                        _GDN,
                        (1,),
                        mode=lax.GatherScatterMode.PROMISE_IN_BOUNDS,
                    )
                    o = out_v[pl.ds(g * 16, 16)]
                    out_v[pl.ds(g * 16, 16)] = jnp.where(
                        lanes16 == t, w, o
                    )

    fire(0, buf_a, sem_a)
    fire(1, buf_b, sem_b)

    @pl.loop(0, NCH // 2 - 1)
    def _pipeline(k):
        n0 = 2 * k
        drain(buf_a, sem_a)
        extract(n0, buf_a)
        fire(n0 + 2, buf_a, sem_a)
        drain(buf_b, sem_b)
        extract(n0 + 1, buf_b)
        fire(n0 + 3, buf_b, sem_b)

    drain(buf_a, sem_a)
    extract(NCH - 2, buf_a)
    drain(buf_b, sem_b)
    extract(NCH - 1, buf_b)

    pltpu.sync_copy(out_v, out_hbm.at[pl.ds(ebase, EPW)])


def kernel(x, index):
    out = _gather_kernel(x, index.reshape(N))
    return out.reshape(ROWS, K)


# speedup vs baseline: 2.0128x; 1.0016x over previous
import functools

import jax
import jax.numpy as jnp
from jax import lax
from jax.experimental import pallas as pl
from jax.experimental.pallas import tpu as pltpu
from jax.experimental.pallas import tpu_sc as plsc

ROWS = 1024
COLS = 100000
K = 64
N = ROWS * K

_info = plsc.get_sparse_core_info()
NC = _info.num_cores
NS = _info.num_subcores
NW = NC * NS
RPW = ROWS // NW
EPW = N // NW
SPW = RPW // 8

CT = 49
CW = CT * 128
NCK = 16
TMAX = (COLS + 127) // 128 - CT
NCH = SPW * NCK

_GDN = lax.GatherDimensionNumbers(
    offset_dims=(), collapsed_slice_dims=(0,), start_index_map=(0,)
)

_mesh = plsc.VectorSubcoreMesh(core_axis_name="c", subcore_axis_name="s")


@functools.partial(
    pl.kernel,
    mesh=_mesh,
    compiler_params=pltpu.CompilerParams(use_tc_tiling_on_sc=True),
    out_type=jax.ShapeDtypeStruct((N,), jnp.float32),
    scratch_types=[
        pltpu.VMEM((EPW,), jnp.int32),
        pltpu.VMEM((EPW,), jnp.float32),
        pltpu.VMEM((8, CW), jnp.float32),
        pltpu.VMEM((8, CW), jnp.float32),
        pltpu.SemaphoreType.DMA,
        pltpu.SemaphoreType.DMA,
    ],
)
def _gather_kernel(
    x_hbm, idx_hbm, out_hbm, idx_v, out_v, buf_a, buf_b, sem_a, sem_b
):
    wid = lax.axis_index("s") * NC + lax.axis_index("c")
    ebase = wid * EPW
    rowbase = wid * RPW
    pltpu.sync_copy(idx_hbm.at[pl.ds(ebase, EPW)], idx_v)

    zeros16 = jnp.zeros((16,), jnp.float32)

    @pl.loop(0, EPW // 16)
    def _init(g):
        out_v[pl.ds(g * 16, 16)] = zeros16

    lanes16 = lax.iota(jnp.int32, 16)

    def fire(n, buf, sem):
        stripe = n >> 4
        t0 = jnp.minimum((n & 15) * CT, TMAX)
        row8 = pl.multiple_of(rowbase + stripe * 8, 8)
        c0 = pl.multiple_of(t0 * 128, 128)
        pltpu.async_copy(
            x_hbm.at[pl.ds(row8, 8), pl.ds(c0, CW)], buf, sem
        )

    def drain(buf, sem):
        pltpu.make_async_copy(
            x_hbm.at[pl.ds(0, 8), pl.ds(0, CW)], buf, sem
        ).wait()

    def extract(n, buf):
        stripe = n >> 4
        col0 = jnp.minimum((n & 15) * CT, TMAX) * 128
        gbase = stripe * 32

        @pl.loop(0, 32)
        def _group(gg):
            subl = (gg >> 2) & 7
            g = gbase + gg
            jv = idx_v[pl.ds(g * 16, 16)]
            for t in range(16):
                s = jv[t]
                loc = s - col0

                @pl.when((loc >= 0) & (loc < CW))
                def _do_elem():
                    v2 = buf[
                        subl, pl.ds(pl.multiple_of(loc & -16, 16), 16)
                    ]
                    w = lax.gather(
                        v2,
                        jnp.full((16,), loc & 15, jnp.int32)[:, None],
                        _GDN,
                        (1,),
                        mode=lax.GatherScatterMode.PROMISE_IN_BOUNDS,
                    )
                    o = out_v[pl.ds(g * 16, 16)]
                    out_v[pl.ds(g * 16, 16)] = jnp.where(
                        lanes16 == t, w, o
                    )

    fire(0, buf_a, sem_a)
    fire(1, buf_b, sem_b)

    @pl.loop(0, NCH // 2 - 1)
    def _pipeline(k):
        n0 = 2 * k
        drain(buf_a, sem_a)
        extract(n0, buf_a)
        fire(n0 + 2, buf_a, sem_a)
        drain(buf_b, sem_b)
        extract(n0 + 1, buf_b)
        fire(n0 + 3, buf_b, sem_b)

    drain(buf_a, sem_a)
    extract(NCH - 2, buf_a)
    drain(buf_b, sem_b)
    extract(NCH - 1, buf_b)

    pltpu.sync_copy(out_v, out_hbm.at[pl.ds(ebase, EPW)])


def kernel(x, index):
    out = _gather_kernel(x, index.reshape(N))
    return out.reshape(ROWS, K)


# speedup vs baseline: 2.4215x; 1.2030x over previous
import functools

import jax
import jax.numpy as jnp
from jax import lax
from jax.experimental import pallas as pl
from jax.experimental.pallas import tpu as pltpu
from jax.experimental.pallas import tpu_sc as plsc

ROWS = 1024
COLS = 100000
K = 64
N = ROWS * K

_info = plsc.get_sparse_core_info()
NC = _info.num_cores
NS = _info.num_subcores
NW = NC * NS
RPW = ROWS // NW
EPW = N // NW

NB = 32
NR = EPW // NB

_GDN = lax.GatherDimensionNumbers(
    offset_dims=(), collapsed_slice_dims=(0,), start_index_map=(0,)
)

_mesh = plsc.VectorSubcoreMesh(core_axis_name="c", subcore_axis_name="s")


@functools.partial(
    pl.kernel,
    mesh=_mesh,
    out_type=jax.ShapeDtypeStruct((N,), jnp.float32),
    scratch_types=[
        pltpu.VMEM((EPW,), jnp.int32),
        pltpu.VMEM((EPW,), jnp.float32),
        pltpu.VMEM((NB * 8, 128), jnp.float32),
        pltpu.VMEM((NB * 8, 128), jnp.float32),
        pltpu.SemaphoreType.DMA,
        pltpu.SemaphoreType.DMA,
    ],
)
def _gather_kernel(
    x_hbm, idx_hbm, out_hbm, idx_v, out_v, buf_a, buf_b, sem_a, sem_b
):
    wid = lax.axis_index("s") * NC + lax.axis_index("c")
    ebase = wid * EPW
    rowbase = wid * RPW
    pltpu.sync_copy(idx_hbm.at[pl.ds(ebase, EPW)], idx_v)

    lanes16 = lax.iota(jnp.int32, 16)

    def fire(n, buf, sem):
        row8 = pl.multiple_of((rowbase + (n >> 1)) & -8, 8)
        for h in range(2):
            jv = idx_v[pl.ds(n * NB + h * 16, 16)]
            cv = jv & -128
            for t in range(16):
                c128 = pl.multiple_of(cv[t], 128)
                pltpu.async_copy(
                    x_hbm.at[pl.ds(row8, 8), pl.ds(c128, 128)],
                    buf.at[pl.ds((h * 16 + t) * 8, 8)],
                    sem,
                )

    def drain(buf, sem):
        pltpu.make_async_copy(
            x_hbm.at[pl.ds(0, NB * 8), pl.ds(0, 128)], buf, sem
        ).wait()

    def extract(n, buf):
        subl = (n >> 1) & 7
        for h in range(2):
            jv = idx_v[pl.ds(n * NB + h * 16, 16)]
            acc = jnp.zeros((16,), jnp.float32)
            for t in range(16):
                s = jv[t]
                v2 = buf[(h * 16 + t) * 8 + subl, pl.ds(s & 112, 16)]
                lvec = jnp.full((16,), s & 15, jnp.int32)
                w = lax.gather(
                    v2,
                    lvec[:, None],
                    _GDN,
                    (1,),
                    mode=lax.GatherScatterMode.PROMISE_IN_BOUNDS,
                )
                acc = jnp.where(lanes16 == t, w, acc)
            out_v[pl.ds(n * NB + h * 16, 16)] = acc

    fire(0, buf_a, sem_a)
    fire(1, buf_b, sem_b)

    @pl.loop(0, NR // 2 - 1)
    def _pipeline(k):
        n0 = 2 * k
        drain(buf_a, sem_a)
        extract(n0, buf_a)
        fire(n0 + 2, buf_a, sem_a)
        drain(buf_b, sem_b)
        extract(n0 + 1, buf_b)
        fire(n0 + 3, buf_b, sem_b)

    drain(buf_a, sem_a)
    extract(NR - 2, buf_a)
    drain(buf_b, sem_b)
    extract(NR - 1, buf_b)

    pltpu.sync_copy(out_v, out_hbm.at[pl.ds(ebase, EPW)])


def kernel(x, index):
    out = _gather_kernel(x, index.reshape(N))
    return out.reshape(ROWS, K)
